# baseline (speedup 1.0000x reference)
import functools
import jax
import jax.numpy as jnp
from jax import lax
from jax.experimental import pallas as pl
from jax.experimental.pallas import tpu as pltpu, tpu_sc as plsc

N_HASHES = 4
CHUNK = 128
HASH_BUCKETS = 32

_NB = 4
_L = 4096
_M = N_HASHES * _L
_NKEY = 160
_NKV = _NKEY // 16
_CE = 64
_C = 256
_NK = _L // CHUNK
_NTILE = 32
_RPT = (_NB * _M) // _NTILE
_GCH = 128

def _pack_bf16_pair(y):
    u = pltpu.bitcast(y.astype(jnp.bfloat16), jnp.uint16)
    w = y.shape[-1] // 2
    lo = u[:, :w].astype(jnp.uint32)
    hi = u[:, w:].astype(jnp.uint32)
    return pltpu.bitcast((hi << 16) | lo, jnp.int32)


def _unpack_bf16_pair(wrd):
    u = pltpu.bitcast(wrd, jnp.uint32)
    lo = (u & 0xFFFF).astype(jnp.uint16)
    hi = (u >> 16).astype(jnp.uint16)
    return jnp.concatenate([pltpu.bitcast(lo, jnp.bfloat16),
                            pltpu.bitcast(hi, jnp.bfloat16)], axis=1)


def _sc_mesh_args():
    return dict(
        mesh=plsc.VectorSubcoreMesh(core_axis_name="c", subcore_axis_name="s"),
        compiler_params=pltpu.CompilerParams(needs_layout_passes=False),
    )


@functools.lru_cache(maxsize=None)
def _make_sc_sort_gather(nb=_NB):
    gch = 64
    bpc = nb // 2
    tpb = 16 // bpc
    rpt = (_M * bpc) // 16
    nch = rpt // gch

    @functools.partial(
        pl.kernel,
        out_type=(
            jax.ShapeDtypeStruct((nb, _M), jnp.int32),
            jax.ShapeDtypeStruct((nb, _M), jnp.int32),
            jax.ShapeDtypeStruct((nb * _M, 2 * _CE), jnp.float32),
            jax.ShapeDtypeStruct((nb * _M, _C // 2), jnp.int32),
        ),
        scratch_types=[
            pltpu.VMEM((_M,), jnp.int32),
            pltpu.VMEM((_M,), jnp.int32),
            pltpu.VMEM((_M,), jnp.int32),
            pltpu.VMEM((_NKEY,), jnp.int32),
            pltpu.VMEM((rpt,), jnp.int32),
            pltpu.VMEM((2, gch, 2 * _CE), jnp.float32),
            pltpu.VMEM((2, gch, _C // 2), jnp.int32),
            pltpu.SemaphoreType.DMA,
            pltpu.SemaphoreType.DMA,
            pltpu.SemaphoreType.DMA,
            pltpu.SemaphoreType.DMA,
        ],
        **_sc_mesh_args(),
    )
    def sc_sort_gather(codes_hbm, xe_hbm, ye_hbm,
                       fwd_hbm, back_hbm, xs_hbm, ys_hbm,
                       codes_v, fwd_v, back_v, table_v,
                       idx_v, xr_v, yr_v, sgx, sgy, ssx, ssy):
        cc = lax.axis_index("c")
        ss_ = lax.axis_index("s")

        @pl.when(ss_ < bpc)
        def _():
            b = bpc * cc + ss_
            pltpu.sync_copy(codes_hbm.at[b], codes_v)
            ones = jnp.ones((16,), jnp.int32)
            for j in range(_NKV):
                table_v[pl.ds(j * 16, 16)] = jnp.zeros((16,), jnp.int32)

            def hist_body(i, carry):
                v = codes_v[pl.ds(i * 16, 16)]
                plsc.addupdate_scatter(table_v, [v], ones)
                return carry

            lax.fori_loop(0, _M // 16, hist_body, 0)

            carry = jnp.zeros((), jnp.int32)
            for j in range(_NKV):
                t = table_v[pl.ds(j * 16, 16)]
                inc = plsc.cumsum(t)
                table_v[pl.ds(j * 16, 16)] = inc - t + carry
                carry = carry + jnp.sum(t)

            iota = lax.iota(jnp.int32, 16)

            def rank_body(i, carry):
                v = codes_v[pl.ds(i * 16, 16)]
                base = plsc.load_gather(table_v, [v])
                within, _ = plsc.scan_count(v)
                rank = base + within - 1
                back_v[pl.ds(i * 16, 16)] = rank + b * _M
                src = (iota + i * 16) % _L + b * _L
                plsc.store_scatter(fwd_v, [rank], src)
                plsc.addupdate_scatter(table_v, [v], ones)
                return carry

            lax.fori_loop(0, _M // 16, rank_body, 0)
            pltpu.sync_copy(fwd_v, fwd_hbm.at[b])
            pltpu.sync_copy(back_v, back_hbm.at[b])

        plsc.subcore_barrier()

        b = bpc * cc + ss_ // tpb
        j = ss_ % tpb
        pltpu.sync_copy(fwd_hbm.at[b, pl.ds(j * rpt, rpt)], idx_v)
        row0 = b * _M + j * rpt
        hg = {}
        hs = {}

        def start_gather(i):
            bi = i % 2
            ival = idx_v.at[pl.ds(i * gch, gch)]
            hg[i] = (
                pltpu.async_copy(xe_hbm.at[ival], xr_v.at[bi], sgx),
                pltpu.async_copy(ye_hbm.at[ival], yr_v.at[bi], sgy),
            )

        start_gather(0)
        for i in range(nch):
            bi = i % 2
            hg[i][0].wait()
            hg[i][1].wait()
            if i + 1 < nch:
                if i >= 1:
                    hs[i - 1][0].wait()
                    hs[i - 1][1].wait()
                start_gather(i + 1)
            out0 = row0 + i * gch
            hs[i] = (
                pltpu.async_copy(xr_v.at[bi], xs_hbm.at[pl.ds(out0, gch)],
                                 ssx),
                pltpu.async_copy(yr_v.at[bi], ys_hbm.at[pl.ds(out0, gch)],
                                 ssy),
            )
        for i in (nch - 2, nch - 1):
            hs[i][0].wait()
            hs[i][1].wait()

    return sc_sort_gather


@functools.lru_cache(maxsize=None)
def _make_sc_gather_back(nb=_NB):
    bpc = nb // 2
    tpb = 16 // bpc
    rpt = (_M * bpc) // 16
    nch = rpt // _GCH

    @functools.partial(
        pl.kernel,
        out_type=(
            jax.ShapeDtypeStruct((nb * _M, _C // 2), jnp.int32),
            jax.ShapeDtypeStruct((nb, _M), jnp.float32),
        ),
        scratch_types=[
            pltpu.VMEM((nch, _GCH), jnp.int32),
            pltpu.VMEM((2, _GCH, _C // 2), jnp.int32),
            pltpu.VMEM((_M,), jnp.float32),
            pltpu.VMEM((rpt,), jnp.float32),
            pltpu.SemaphoreType.DMA,
            pltpu.SemaphoreType.DMA,
        ],
        **_sc_mesh_args(),
    )
    def gback(idx_hbm, ret_hbm, score_hbm, retg_hbm, scoreg_hbm,
              idx_v, rr_v, stab_v, sout_v, sg, ss):
        cc = lax.axis_index("c")
        ss_ = lax.axis_index("s")
        b = bpc * cc + ss_ // tpb
        j = ss_ % tpb
        pltpu.sync_copy(idx_hbm.at[b, pl.ds(j * nch, nch)], idx_v)
        pltpu.sync_copy(score_hbm.at[b], stab_v)
        row0 = b * _M + j * rpt
        boff = b * _M
        hg = {}
        hs = {}
        hg[0] = pltpu.async_copy(ret_hbm.at[idx_v.at[0]], rr_v.at[0], sg)
        for i in range(nch):
            bi = i % 2
            if i + 1 < nch:
                if i >= 1:
                    hs[i - 1].wait()
                hg[i + 1] = pltpu.async_copy(
                    ret_hbm.at[idx_v.at[i + 1]], rr_v.at[1 - bi], sg)
            for l in range(_GCH // 16):
                v = idx_v[i, pl.ds(l * 16, 16)] - boff
                sout_v[pl.ds(i * _GCH + l * 16, 16)] = \
                    plsc.load_gather(stab_v, [v])
            hg[i].wait()
            hs[i] = pltpu.async_copy(
                rr_v.at[bi], retg_hbm.at[pl.ds(row0 + i * _GCH, _GCH)], ss)
        hs[nch - 2].wait()
        hs[nch - 1].wait()
        pltpu.sync_copy(sout_v, scoreg_hbm.at[b, pl.ds(j * rpt, rpt)])

    return gback


def _embed_body(x_ref, xm_ref, xp_ref, w0_ref, w1_ref, w2_ref, wa_ref,
                ba_ref, rot_ref, xe_ref, ye_ref, code_ref):
    pid = pl.program_id(1)
    nsp = pl.num_programs(1)
    x = x_ref[0]
    T = x.shape[0]
    dot = lambda a, b: jax.lax.dot_general(
        a, b, (((1,), (0,)), ((), ())), preferred_element_type=jnp.float32)
    xprev = jnp.concatenate([xm_ref[0, 7:8, :], x[:-1]], axis=0)
    xnext = jnp.concatenate([x[1:], xp_ref[0, 0:1, :]], axis=0)
    e0 = dot(xprev, w0_ref[...])
    e2 = dot(xnext, w2_ref[...])
    riota = jax.lax.broadcasted_iota(jnp.int32, (T, 1), 0)
    e0 = jnp.where((pid == 0) & (riota == 0), 0.0, e0)
    e2 = jnp.where((pid == nsp - 1) & (riota == T - 1), 0.0, e2)
    xe = e0 + dot(x, w1_ref[...]) + e2
    nrm = jnp.sqrt(jnp.sum(xe * xe, axis=-1, keepdims=True))
    xn = xe / jnp.maximum(nrm, 5e-05)
    xe_ref[0] = jnp.concatenate([xe, xn], axis=1)
    ye = jax.lax.dot_general(
        x, wa_ref[...], (((1,), (0,)), ((), ())),
        preferred_element_type=jnp.float32) + ba_ref[...][None, :]
    ye_ref[0] = _pack_bf16_pair(ye)

    rot = dot(xe, rot_ref[...])
    iota = jax.lax.broadcasted_iota(jnp.int32, (T, HASH_BUCKETS), 1)
    big = jnp.int32(2 * HASH_BUCKETS)
    for h in range(N_HASHES):
        a = rot[:, h * HASH_BUCKETS:(h + 1) * HASH_BUCKETS]
        m1 = jnp.max(a, axis=1)
        i1 = jnp.min(jnp.where(a == m1[:, None], iota, big), axis=1)
        na = -a
        m2 = jnp.max(na, axis=1)
        i2 = HASH_BUCKETS + jnp.min(jnp.where(na == m2[:, None], iota, big),
                                    axis=1)
        code = jnp.where(m1 >= m2, i1, i2) + h * HASH_BUCKETS
        code_ref[0, h] = code


def _embed_hash(x, w_match, w_assembly, b_assembly, random_rotations,
                interpret=False):
    N, L, C = x.shape
    T = 2048
    nsp = L // T
    w0, w1, w2 = (w_match[:, :, k].T for k in range(3))
    wa = w_assembly[:, :, 0].T
    rot = random_rotations[0].reshape(_CE, N_HASHES * HASH_BUCKETS)
    nrb = T // 8
    xe, ye, codes = pl.pallas_call(
        _embed_body,
        grid=(N, nsp),
        in_specs=[
            pl.BlockSpec((1, T, C), lambda b, t: (b, t, 0)),
            pl.BlockSpec((1, 8, C),
                         lambda b, t: (b, jnp.maximum(t * nrb - 1, 0), 0)),
            pl.BlockSpec((1, 8, C),
                         lambda b, t: (b, jnp.minimum((t + 1) * nrb,
                                                      L // 8 - 1), 0)),
            pl.BlockSpec((C, _CE), lambda b, t: (0, 0)),
            pl.BlockSpec((C, _CE), lambda b, t: (0, 0)),
            pl.BlockSpec((C, _CE), lambda b, t: (0, 0)),
            pl.BlockSpec((C, C), lambda b, t: (0, 0)),
            pl.BlockSpec((C,), lambda b, t: (0,)),
            pl.BlockSpec((_CE, N_HASHES * HASH_BUCKETS), lambda b, t: (0, 0)),
        ],
        out_specs=(
            pl.BlockSpec((1, T, 2 * _CE), lambda b, t: (b, t, 0)),
            pl.BlockSpec((1, T, C // 2), lambda b, t: (b, t, 0)),
            pl.BlockSpec((1, N_HASHES, T), lambda b, t: (b, 0, t)),
        ),
        out_shape=(
            jax.ShapeDtypeStruct((N, L, 2 * _CE), jnp.float32),
            jax.ShapeDtypeStruct((N, L, C // 2), jnp.int32),
            jax.ShapeDtypeStruct((N, N_HASHES, L), jnp.int32),
        ),
        compiler_params=pltpu.CompilerParams(
            vmem_limit_bytes=100 * 1024 * 1024),
        interpret=interpret,
    )(x, x, x, w0, w1, w2, wa, b_assembly, rot)
    return xe, ye, codes


_KPG = 16


def _attn_body(x_ref, y_ref, ret_ref, score_ref):
    g = pl.program_id(2)

    def one(kk):
        km1 = (kk + _NK - 1) % _NK
        kp1 = (kk + 1) % _NK
        xq = x_ref[0, 0, kk]
        q = xq[:, :_CE]
        kcat = jnp.concatenate(
            [xq[:, _CE:], x_ref[0, 0, km1][:, _CE:],
             x_ref[0, 0, kp1][:, _CE:]], axis=0)
        raw = jax.lax.dot_general(
            q, kcat, (((1,), (1,)), ((), ())),
            preferred_element_type=jnp.float32)
        m = jnp.max(raw, axis=-1, keepdims=True)
        e = jnp.exp(raw - m)
        s = jnp.sum(e, axis=-1, keepdims=True)
        p = e / s
        unpack = lambda yi: _unpack_bf16_pair(yi)
        ycat = jnp.concatenate(
            [unpack(y_ref[0, 0, kk]), unpack(y_ref[0, 0, km1]),
             unpack(y_ref[0, 0, kp1])], axis=0)
        ret = jax.lax.dot_general(
            p.astype(jnp.bfloat16), ycat, (((1,), (0,)), ((), ())),
            preferred_element_type=jnp.float32)
        return _pack_bf16_pair(ret), (m + jnp.log(s))[:, 0]

    for kc in range(_KPG):
        ret, sc = one(g * _KPG + kc)
        ret_ref[0, 0, kc] = ret
        score_ref[0, 0, kc, 0] = sc


def _attention(x_s, y_s, interpret=False):
    N, H = x_s.shape[0], x_s.shape[1]
    grid = (N, H, _NK // _KPG)
    out_shapes = (
        jax.ShapeDtypeStruct((N, H, _NK, CHUNK, _C // 2), jnp.int32),
        jax.ShapeDtypeStruct((N, H, _NK, 1, CHUNK), jnp.float32),
    )
    out_specs = (
        pl.BlockSpec((1, 1, _KPG, CHUNK, _C // 2),
                     lambda b, h, g: (b, h, g, 0, 0)),
        pl.BlockSpec((1, 1, _KPG, 1, CHUNK),
                     lambda b, h, g: (b, h, g, 0, 0)),
    )
    ret, score = pl.pallas_call(
        _attn_body,
        grid=grid,
        in_specs=[
            pl.BlockSpec((1, 1, _NK, CHUNK, x_s.shape[-1]),
                         lambda b, h, g: (b, h, 0, 0, 0)),
            pl.BlockSpec((1, 1, _NK, CHUNK, _C // 2),
                         lambda b, h, g: (b, h, 0, 0, 0)),
        ],
        out_specs=out_specs,
        out_shape=out_shapes,
        compiler_params=pltpu.CompilerParams(
            vmem_limit_bytes=100 * 1024 * 1024),
        interpret=interpret,
    )(x_s, y_s)
    return ret, score


def _combine_body(score_ref, ret_ref, x_ref, out_ref):
    s = score_ref[0]
    m = jnp.max(s, axis=0, keepdims=True)
    e = jnp.exp(s - m)
    p = e / jnp.sum(e, axis=0, keepdims=True)
    acc = x_ref[0]
    T = acc.shape[0]
    for r in range(N_HASHES):
        rb = _unpack_bf16_pair(ret_ref[0, r]).astype(jnp.float32)
        acc = acc + p[r][:, None] * rb
    out_ref[0] = acc


def _combine(score_g, ret_g, x, interpret=False):
    N, H, L = score_g.shape
    C = x.shape[-1]
    T = 2048
    grid = (N, L // T)
    out = pl.pallas_call(
        _combine_body,
        grid=grid,
        in_specs=[
            pl.BlockSpec((1, H, T), lambda b, t: (b, 0, t)),
            pl.BlockSpec((1, H, T, C // 2), lambda b, t: (b, 0, t, 0)),
            pl.BlockSpec((1, T, C), lambda b, t: (b, t, 0)),
        ],
        out_specs=pl.BlockSpec((1, T, C), lambda b, t: (b, t, 0)),
        out_shape=jax.ShapeDtypeStruct((N, L, C), jnp.float32),
        compiler_params=pltpu.CompilerParams(
            vmem_limit_bytes=100 * 1024 * 1024),
        interpret=interpret,
    )(score_g, ret_g, x)
    return out


def kernel(input, w_match, w_assembly, b_assembly, random_rotations,
           interpret=False):
    x = input
    N, L, C = x.shape
    Ce = _CE

    def pipeline(xg):
        ng = xg.shape[0]
        xe_pad, y_embed, codes = _embed_hash(
            xg, w_match, w_assembly, b_assembly, random_rotations,
            interpret=interpret)
        hash_codes = codes.reshape(ng, -1)

        if interpret:
            indices = jnp.argsort(hash_codes, axis=-1)
            undo_sort = jnp.argsort(indices, axis=-1)
            mod_indices = indices % L
            x_sorted = jnp.take_along_axis(
                xe_pad.reshape(ng, L, 2 * Ce), mod_indices[:, :, None],
                axis=1)
            y_sorted = jnp.take_along_axis(
                y_embed, mod_indices[:, :, None], axis=1)
        else:
            fwd_ids, back_ids, x_sorted, y_sorted = \
                _make_sc_sort_gather(ng)(
                    hash_codes,
                    xe_pad.reshape(ng * L, 2 * Ce),
                    y_embed.reshape(ng * L, C // 2),
                )

        x_att = x_sorted.reshape(ng, N_HASHES, _NK, CHUNK,
                                 x_sorted.shape[-1])
        y_att = y_sorted.reshape(ng, N_HASHES, _NK, CHUNK, C // 2)

        ret, score = _attention(x_att, y_att, interpret=interpret)

        ret = ret.reshape(ng * N_HASHES * L, C // 2)
        score = score.reshape(ng, N_HASHES * L)
        if interpret:
            ret_g = jnp.take_along_axis(
                ret.reshape(ng, N_HASHES * L, C // 2),
                undo_sort[:, :, None], axis=1)
            score_g = jnp.take_along_axis(score, undo_sort, axis=1)
        else:
            ret_g, score_g = _make_sc_gather_back(ng)(
                back_ids.reshape(ng, _M // _GCH, _GCH), ret, score)
        ret_g = ret_g.reshape(ng, N_HASHES, L, C // 2)
        score_g = score_g.reshape(ng, N_HASHES, L)

        return _combine(score_g, ret_g, xg, interpret=interpret)

    half = N // 2
    return jnp.concatenate(
        [pipeline(x[:half]), pipeline(x[half:])], axis=0)


# speedup vs baseline: 1.0664x; 1.0664x over previous
import functools
import jax
import jax.numpy as jnp
from jax import lax
from jax.experimental import pallas as pl
from jax.experimental.pallas import tpu as pltpu, tpu_sc as plsc

N_HASHES = 4
CHUNK = 128
HASH_BUCKETS = 32

_NB = 4
_L = 4096
_M = N_HASHES * _L
_NKEY = 160
_NKV = _NKEY // 16
_CE = 64
_C = 256
_NK = _L // CHUNK
_NTILE = 32
_RPT = (_NB * _M) // _NTILE
_GCH = 128

def _pack_bf16_pair(y):
    u = pltpu.bitcast(y.astype(jnp.bfloat16), jnp.uint16)
    w = y.shape[-1] // 2
    lo = u[:, :w].astype(jnp.uint32)
    hi = u[:, w:].astype(jnp.uint32)
    return pltpu.bitcast((hi << 16) | lo, jnp.int32)


def _unpack_bf16_pair(wrd):
    u = pltpu.bitcast(wrd, jnp.uint32)
    lo = (u & 0xFFFF).astype(jnp.uint16)
    hi = (u >> 16).astype(jnp.uint16)
    return jnp.concatenate([pltpu.bitcast(lo, jnp.bfloat16),
                            pltpu.bitcast(hi, jnp.bfloat16)], axis=1)


def _sc_mesh_args():
    return dict(
        mesh=plsc.VectorSubcoreMesh(core_axis_name="c", subcore_axis_name="s"),
        compiler_params=pltpu.CompilerParams(needs_layout_passes=False),
    )


@functools.lru_cache(maxsize=None)
def _make_sc_sort_gather(nb=_NB):
    gch = 64
    bpc = nb // 2
    tpb = 16 // bpc
    rpt = (_M * bpc) // 16
    nch = rpt // gch

    @functools.partial(
        pl.kernel,
        out_type=(
            jax.ShapeDtypeStruct((nb, _M), jnp.int32),
            jax.ShapeDtypeStruct((nb, _M), jnp.int32),
            jax.ShapeDtypeStruct((nb * _M, 2 * _CE), jnp.float32),
            jax.ShapeDtypeStruct((nb * _M, _C // 2), jnp.int32),
        ),
        scratch_types=[
            pltpu.VMEM((_M,), jnp.int32),
            pltpu.VMEM((_M,), jnp.int32),
            pltpu.VMEM((_M,), jnp.int32),
            pltpu.VMEM((_NKEY,), jnp.int32),
            pltpu.VMEM((rpt,), jnp.int32),
            pltpu.VMEM((2, gch, 2 * _CE), jnp.float32),
            pltpu.VMEM((2, gch, _C // 2), jnp.int32),
            pltpu.SemaphoreType.DMA,
            pltpu.SemaphoreType.DMA,
            pltpu.SemaphoreType.DMA,
            pltpu.SemaphoreType.DMA,
        ],
        **_sc_mesh_args(),
    )
    def sc_sort_gather(codes_hbm, xe_hbm, ye_hbm,
                       fwd_hbm, back_hbm, xs_hbm, ys_hbm,
                       codes_v, fwd_v, back_v, table_v,
                       idx_v, xr_v, yr_v, sgx, sgy, ssx, ssy):
        cc = lax.axis_index("c")
        ss_ = lax.axis_index("s")

        @pl.when(ss_ < bpc)
        def _():
            b = bpc * cc + ss_
            pltpu.sync_copy(codes_hbm.at[b], codes_v)
            ones = jnp.ones((16,), jnp.int32)
            for j in range(_NKV):
                table_v[pl.ds(j * 16, 16)] = jnp.zeros((16,), jnp.int32)

            def hist_body(i, carry):
                v = codes_v[pl.ds(i * 16, 16)]
                plsc.addupdate_scatter(table_v, [v], ones)
                return carry

            lax.fori_loop(0, _M // 16, hist_body, 0)

            carry = jnp.zeros((), jnp.int32)
            for j in range(_NKV):
                t = table_v[pl.ds(j * 16, 16)]
                inc = plsc.cumsum(t)
                table_v[pl.ds(j * 16, 16)] = inc - t + carry
                carry = carry + jnp.sum(t)

            iota = lax.iota(jnp.int32, 16)

            def rank_body(i, carry):
                v = codes_v[pl.ds(i * 16, 16)]
                base = plsc.load_gather(table_v, [v])
                within, _ = plsc.scan_count(v)
                rank = base + within - 1
                back_v[pl.ds(i * 16, 16)] = rank + b * _M
                src = (iota + i * 16) % _L + b * _L
                plsc.store_scatter(fwd_v, [rank], src)
                plsc.addupdate_scatter(table_v, [v], ones)
                return carry

            lax.fori_loop(0, _M // 16, rank_body, 0)
            pltpu.sync_copy(fwd_v, fwd_hbm.at[b])
            pltpu.sync_copy(back_v, back_hbm.at[b])

        plsc.subcore_barrier()

        b = bpc * cc + ss_ // tpb
        j = ss_ % tpb
        pltpu.sync_copy(fwd_hbm.at[b, pl.ds(j * rpt, rpt)], idx_v)
        row0 = b * _M + j * rpt
        hg = {}
        hs = {}

        def start_gather(i):
            bi = i % 2
            ival = idx_v.at[pl.ds(i * gch, gch)]
            hg[i] = (
                pltpu.async_copy(xe_hbm.at[ival], xr_v.at[bi], sgx),
                pltpu.async_copy(ye_hbm.at[ival], yr_v.at[bi], sgy),
            )

        start_gather(0)
        for i in range(nch):
            bi = i % 2
            hg[i][0].wait()
            hg[i][1].wait()
            if i + 1 < nch:
                if i >= 1:
                    hs[i - 1][0].wait()
                    hs[i - 1][1].wait()
                start_gather(i + 1)
            out0 = row0 + i * gch
            hs[i] = (
                pltpu.async_copy(xr_v.at[bi], xs_hbm.at[pl.ds(out0, gch)],
                                 ssx),
                pltpu.async_copy(yr_v.at[bi], ys_hbm.at[pl.ds(out0, gch)],
                                 ssy),
            )
        for i in (nch - 2, nch - 1):
            hs[i][0].wait()
            hs[i][1].wait()

    return sc_sort_gather


@functools.lru_cache(maxsize=None)
def _make_sc_gather_back(nb=_NB):
    bpc = nb // 2
    tpb = 16 // bpc
    rpt = (_M * bpc) // 16
    nch = rpt // _GCH

    @functools.partial(
        pl.kernel,
        out_type=(
            jax.ShapeDtypeStruct((nb * _M, _C // 2), jnp.int32),
            jax.ShapeDtypeStruct((nb, _M), jnp.float32),
        ),
        scratch_types=[
            pltpu.VMEM((nch, _GCH), jnp.int32),
            pltpu.VMEM((2, _GCH, _C // 2), jnp.int32),
            pltpu.VMEM((_M,), jnp.float32),
            pltpu.VMEM((rpt,), jnp.float32),
            pltpu.SemaphoreType.DMA,
            pltpu.SemaphoreType.DMA,
        ],
        **_sc_mesh_args(),
    )
    def gback(idx_hbm, ret_hbm, score_hbm, retg_hbm, scoreg_hbm,
              idx_v, rr_v, stab_v, sout_v, sg, ss):
        cc = lax.axis_index("c")
        ss_ = lax.axis_index("s")
        b = bpc * cc + ss_ // tpb
        j = ss_ % tpb
        pltpu.sync_copy(idx_hbm.at[b, pl.ds(j * nch, nch)], idx_v)
        pltpu.sync_copy(score_hbm.at[b], stab_v)
        row0 = b * _M + j * rpt
        boff = b * _M
        hg = {}
        hs = {}
        hg[0] = pltpu.async_copy(ret_hbm.at[idx_v.at[0]], rr_v.at[0], sg)
        for i in range(nch):
            bi = i % 2
            if i + 1 < nch:
                if i >= 1:
                    hs[i - 1].wait()
                hg[i + 1] = pltpu.async_copy(
                    ret_hbm.at[idx_v.at[i + 1]], rr_v.at[1 - bi], sg)
            for l in range(_GCH // 16):
                v = idx_v[i, pl.ds(l * 16, 16)] - boff
                sout_v[pl.ds(i * _GCH + l * 16, 16)] = \
                    plsc.load_gather(stab_v, [v])
            hg[i].wait()
            hs[i] = pltpu.async_copy(
                rr_v.at[bi], retg_hbm.at[pl.ds(row0 + i * _GCH, _GCH)], ss)
        hs[nch - 2].wait()
        hs[nch - 1].wait()
        pltpu.sync_copy(sout_v, scoreg_hbm.at[b, pl.ds(j * rpt, rpt)])

    return gback


def _embed_body(x_ref, xm_ref, xp_ref, w0_ref, w1_ref, w2_ref, wa_ref,
                ba_ref, rot_ref, xe_ref, ye_ref, code_ref):
    pid = pl.program_id(1)
    nsp = pl.num_programs(1)
    x = x_ref[0]
    T = x.shape[0]
    dot = lambda a, b: jax.lax.dot_general(
        a, b, (((1,), (0,)), ((), ())), preferred_element_type=jnp.float32)
    xprev = jnp.concatenate([xm_ref[0, 7:8, :], x[:-1]], axis=0)
    xnext = jnp.concatenate([x[1:], xp_ref[0, 0:1, :]], axis=0)
    e0 = dot(xprev, w0_ref[...])
    e2 = dot(xnext, w2_ref[...])
    riota = jax.lax.broadcasted_iota(jnp.int32, (T, 1), 0)
    e0 = jnp.where((pid == 0) & (riota == 0), 0.0, e0)
    e2 = jnp.where((pid == nsp - 1) & (riota == T - 1), 0.0, e2)
    xe = e0 + dot(x, w1_ref[...]) + e2
    nrm = jnp.sqrt(jnp.sum(xe * xe, axis=-1, keepdims=True))
    xn = xe / jnp.maximum(nrm, 5e-05)
    xe_ref[0] = jnp.concatenate([xe, xn], axis=1)
    ye = jax.lax.dot_general(
        x, wa_ref[...], (((1,), (0,)), ((), ())),
        preferred_element_type=jnp.float32) + ba_ref[...][None, :]
    ye_ref[0] = _pack_bf16_pair(ye)

    rot = dot(xe, rot_ref[...])
    iota = jax.lax.broadcasted_iota(jnp.int32, (T, HASH_BUCKETS), 1)
    big = jnp.int32(2 * HASH_BUCKETS)
    for h in range(N_HASHES):
        a = rot[:, h * HASH_BUCKETS:(h + 1) * HASH_BUCKETS]
        m1 = jnp.max(a, axis=1)
        i1 = jnp.min(jnp.where(a == m1[:, None], iota, big), axis=1)
        na = -a
        m2 = jnp.max(na, axis=1)
        i2 = HASH_BUCKETS + jnp.min(jnp.where(na == m2[:, None], iota, big),
                                    axis=1)
        code = jnp.where(m1 >= m2, i1, i2) + h * HASH_BUCKETS
        code_ref[0, h] = code


def _embed_hash(x, w_match, w_assembly, b_assembly, random_rotations,
                interpret=False):
    N, L, C = x.shape
    T = 2048
    nsp = L // T
    w0, w1, w2 = (w_match[:, :, k].T for k in range(3))
    wa = w_assembly[:, :, 0].T
    rot = random_rotations[0].reshape(_CE, N_HASHES * HASH_BUCKETS)
    nrb = T // 8
    xe, ye, codes = pl.pallas_call(
        _embed_body,
        grid=(N, nsp),
        in_specs=[
            pl.BlockSpec((1, T, C), lambda b, t: (b, t, 0)),
            pl.BlockSpec((1, 8, C),
                         lambda b, t: (b, jnp.maximum(t * nrb - 1, 0), 0)),
            pl.BlockSpec((1, 8, C),
                         lambda b, t: (b, jnp.minimum((t + 1) * nrb,
                                                      L // 8 - 1), 0)),
            pl.BlockSpec((C, _CE), lambda b, t: (0, 0)),
            pl.BlockSpec((C, _CE), lambda b, t: (0, 0)),
            pl.BlockSpec((C, _CE), lambda b, t: (0, 0)),
            pl.BlockSpec((C, C), lambda b, t: (0, 0)),
            pl.BlockSpec((C,), lambda b, t: (0,)),
            pl.BlockSpec((_CE, N_HASHES * HASH_BUCKETS), lambda b, t: (0, 0)),
        ],
        out_specs=(
            pl.BlockSpec((1, T, 2 * _CE), lambda b, t: (b, t, 0)),
            pl.BlockSpec((1, T, C // 2), lambda b, t: (b, t, 0)),
            pl.BlockSpec((1, N_HASHES, T), lambda b, t: (b, 0, t)),
        ),
        out_shape=(
            jax.ShapeDtypeStruct((N, L, 2 * _CE), jnp.float32),
            jax.ShapeDtypeStruct((N, L, C // 2), jnp.int32),
            jax.ShapeDtypeStruct((N, N_HASHES, L), jnp.int32),
        ),
        compiler_params=pltpu.CompilerParams(
            vmem_limit_bytes=100 * 1024 * 1024),
        interpret=interpret,
    )(x, x, x, w0, w1, w2, wa, b_assembly, rot)
    return xe, ye, codes


_KPG = 32


def _attn_body(x_ref, y_ref, ret_ref, score_ref):
    g = pl.program_id(2)

    def one(kk):
        km1 = (kk + _NK - 1) % _NK
        kp1 = (kk + 1) % _NK
        xq = x_ref[0, 0, kk]
        q = xq[:, :_CE]
        kcat = jnp.concatenate(
            [xq[:, _CE:], x_ref[0, 0, km1][:, _CE:],
             x_ref[0, 0, kp1][:, _CE:]], axis=0)
        raw = jax.lax.dot_general(
            q, kcat, (((1,), (1,)), ((), ())),
            preferred_element_type=jnp.float32)
        m = jnp.max(raw, axis=-1, keepdims=True)
        e = jnp.exp(raw - m)
        s = jnp.sum(e, axis=-1, keepdims=True)
        p = e / s
        unpack = lambda yi: _unpack_bf16_pair(yi)
        ycat = jnp.concatenate(
            [unpack(y_ref[0, 0, kk]), unpack(y_ref[0, 0, km1]),
             unpack(y_ref[0, 0, kp1])], axis=0)
        ret = jax.lax.dot_general(
            p.astype(jnp.bfloat16), ycat, (((1,), (0,)), ((), ())),
            preferred_element_type=jnp.float32)
        return _pack_bf16_pair(ret), (m + jnp.log(s))[:, 0]

    for kc in range(_KPG):
        ret, sc = one(g * _KPG + kc)
        ret_ref[0, 0, kc] = ret
        score_ref[0, 0, kc, 0] = sc


def _attention(x_s, y_s, interpret=False):
    N, H = x_s.shape[0], x_s.shape[1]
    grid = (N, H, _NK // _KPG)
    out_shapes = (
        jax.ShapeDtypeStruct((N, H, _NK, CHUNK, _C // 2), jnp.int32),
        jax.ShapeDtypeStruct((N, H, _NK, 1, CHUNK), jnp.float32),
    )
    out_specs = (
        pl.BlockSpec((1, 1, _KPG, CHUNK, _C // 2),
                     lambda b, h, g: (b, h, g, 0, 0)),
        pl.BlockSpec((1, 1, _KPG, 1, CHUNK),
                     lambda b, h, g: (b, h, g, 0, 0)),
    )
    ret, score = pl.pallas_call(
        _attn_body,
        grid=grid,
        in_specs=[
            pl.BlockSpec((1, 1, _NK, CHUNK, x_s.shape[-1]),
                         lambda b, h, g: (b, h, 0, 0, 0)),
            pl.BlockSpec((1, 1, _NK, CHUNK, _C // 2),
                         lambda b, h, g: (b, h, 0, 0, 0)),
        ],
        out_specs=out_specs,
        out_shape=out_shapes,
        compiler_params=pltpu.CompilerParams(
            vmem_limit_bytes=100 * 1024 * 1024),
        interpret=interpret,
    )(x_s, y_s)
    return ret, score


def _combine_body(score_ref, ret_ref, x_ref, out_ref):
    s = score_ref[0]
    m = jnp.max(s, axis=0, keepdims=True)
    e = jnp.exp(s - m)
    p = e / jnp.sum(e, axis=0, keepdims=True)
    acc = x_ref[0]
    T = acc.shape[0]
    for r in range(N_HASHES):
        rb = _unpack_bf16_pair(ret_ref[0, r]).astype(jnp.float32)
        acc = acc + p[r][:, None] * rb
    out_ref[0] = acc


def _combine(score_g, ret_g, x, interpret=False):
    N, H, L = score_g.shape
    C = x.shape[-1]
    T = 2048
    grid = (N, L // T)
    out = pl.pallas_call(
        _combine_body,
        grid=grid,
        in_specs=[
            pl.BlockSpec((1, H, T), lambda b, t: (b, 0, t)),
            pl.BlockSpec((1, H, T, C // 2), lambda b, t: (b, 0, t, 0)),
            pl.BlockSpec((1, T, C), lambda b, t: (b, t, 0)),
        ],
        out_specs=pl.BlockSpec((1, T, C), lambda b, t: (b, t, 0)),
        out_shape=jax.ShapeDtypeStruct((N, L, C), jnp.float32),
        compiler_params=pltpu.CompilerParams(
            vmem_limit_bytes=100 * 1024 * 1024),
        interpret=interpret,
    )(score_g, ret_g, x)
    return out


def kernel(input, w_match, w_assembly, b_assembly, random_rotations,
           interpret=False):
    x = input
    N, L, C = x.shape
    Ce = _CE

    def pipeline(xg):
        ng = xg.shape[0]
        xe_pad, y_embed, codes = _embed_hash(
            xg, w_match, w_assembly, b_assembly, random_rotations,
            interpret=interpret)
        hash_codes = codes.reshape(ng, -1)

        if interpret:
            indices = jnp.argsort(hash_codes, axis=-1)
            undo_sort = jnp.argsort(indices, axis=-1)
            mod_indices = indices % L
            x_sorted = jnp.take_along_axis(
                xe_pad.reshape(ng, L, 2 * Ce), mod_indices[:, :, None],
                axis=1)
            y_sorted = jnp.take_along_axis(
                y_embed, mod_indices[:, :, None], axis=1)
        else:
            fwd_ids, back_ids, x_sorted, y_sorted = \
                _make_sc_sort_gather(ng)(
                    hash_codes,
                    xe_pad.reshape(ng * L, 2 * Ce),
                    y_embed.reshape(ng * L, C // 2),
                )

        x_att = x_sorted.reshape(ng, N_HASHES, _NK, CHUNK,
                                 x_sorted.shape[-1])
        y_att = y_sorted.reshape(ng, N_HASHES, _NK, CHUNK, C // 2)

        ret, score = _attention(x_att, y_att, interpret=interpret)

        ret = ret.reshape(ng * N_HASHES * L, C // 2)
        score = score.reshape(ng, N_HASHES * L)
        if interpret:
            ret_g = jnp.take_along_axis(
                ret.reshape(ng, N_HASHES * L, C // 2),
                undo_sort[:, :, None], axis=1)
            score_g = jnp.take_along_axis(score, undo_sort, axis=1)
        else:
            ret_g, score_g = _make_sc_gather_back(ng)(
                back_ids.reshape(ng, _M // _GCH, _GCH), ret, score)
        ret_g = ret_g.reshape(ng, N_HASHES, L, C // 2)
        score_g = score_g.reshape(ng, N_HASHES, L)

        return _combine(score_g, ret_g, xg, interpret=interpret)

    half = N // 2
    return jnp.concatenate(
        [pipeline(x[:half]), pipeline(x[half:])], axis=0)
